# baseline (speedup 1.0000x reference)
import functools

import jax
import jax.numpy as jnp
from jax.experimental import pallas as pl
from jax.experimental.pallas import tpu as pltpu

EPS = 1e-6
BLK = 256


def _rot_perm(hd):
    h = hd // 2
    eye = jnp.eye(h, dtype=jnp.float32)
    z = jnp.zeros((h, h), jnp.float32)
    return jnp.block([[z, eye], [-eye, z]])


def _prelude_body(x_ref, anw_ref, wq_ref, wk_ref, wv_ref, qnw_ref, knw_ref,
                  cq_ref, sq_ref, ck_ref, sk_ref, pq_ref, pk_ref,
                  hq_ref, hqt_ref, hk_ref, hkt_ref,
                  q_ref, k_ref, v_ref, *, hd):
    xs = x_ref[...]
    a = xs * jax.lax.rsqrt(jnp.mean(xs * xs, axis=-1, keepdims=True) + EPS)
    a = (a * anw_ref[...]).astype(jnp.bfloat16)

    def qk_path(w_ref, nw_ref, h_ref, ht_ref, p_ref, c_ref, s_ref):
        q = jnp.dot(a, w_ref[...], preferred_element_type=jnp.float32)
        ss = jnp.dot(q * q, h_ref[...], preferred_element_type=jnp.float32)
        rs = jax.lax.rsqrt(ss / hd + EPS)
        qn = q * jnp.dot(rs, ht_ref[...], preferred_element_type=jnp.float32)
        qn = qn * nw_ref[...]
        qr = jnp.dot(qn.astype(jnp.bfloat16), p_ref[...],
                     preferred_element_type=jnp.float32)
        return (qn * c_ref[...] + qr * s_ref[...]).astype(jnp.bfloat16)

    q_ref[...] = qk_path(wq_ref, qnw_ref, hq_ref, hqt_ref, pq_ref, cq_ref, sq_ref)
    k_ref[...] = qk_path(wk_ref, knw_ref, hk_ref, hkt_ref, pk_ref, ck_ref, sk_ref)
    v_ref[...] = jnp.dot(a, wv_ref[...],
                         preferred_element_type=jnp.float32).astype(jnp.bfloat16)


def _attn_body(q_ref, k_ref, v_ref, o_ref, *, hd):
    s = jax.lax.dot_general(q_ref[0], k_ref[0],
                            (((1,), (1,)), ((), ())),
                            preferred_element_type=jnp.float32)
    s = s * (1.0 / (hd ** 0.5))
    m = jnp.max(s, axis=-1, keepdims=True)
    e = jnp.exp(s - m)
    p = e / jnp.sum(e, axis=-1, keepdims=True)
    o_ref[0] = jnp.dot(p.astype(jnp.bfloat16), v_ref[0],
                       preferred_element_type=jnp.float32).astype(jnp.bfloat16)


def _post_body(ctx_ref, wo_ref, x_ref, fnw_ref, rw_ref,
               x2_ref, m_ref, comb_ref, *, ne):
    x2 = x_ref[...] + jnp.dot(ctx_ref[...], wo_ref[...],
                              preferred_element_type=jnp.float32)
    x2_ref[...] = x2
    mm = x2 * jax.lax.rsqrt(jnp.mean(x2 * x2, axis=-1, keepdims=True) + EPS)
    mm = mm * fnw_ref[...]
    m_ref[...] = mm.astype(jnp.bfloat16)
    logits = jnp.dot(mm, rw_ref[...], preferred_element_type=jnp.float32)
    mx = jnp.max(logits, axis=-1, keepdims=True)
    ex = jnp.exp(logits - mx)
    g = ex / jnp.sum(ex, axis=-1, keepdims=True)
    it = jax.lax.broadcasted_iota(jnp.int32, g.shape, 1)
    m1 = jnp.max(g, axis=-1, keepdims=True)
    i1 = jnp.min(jnp.where(g == m1, it, ne), axis=-1, keepdims=True)
    g2 = jnp.where(it == i1, -jnp.inf, g)
    m2 = jnp.max(g2, axis=-1, keepdims=True)
    i2 = jnp.min(jnp.where(g2 == m2, it, ne), axis=-1, keepdims=True)
    comb_ref[...] = (jnp.where(it == i1, m1, 0.0)
                     + jnp.where(it == i2, m2, 0.0))


def _ffn_body(m_ref, wg_ref, wi_ref, woe_ref, comb_ref, x2_ref, out_ref):
    e = pl.program_id(1)

    @pl.when(e == 0)
    def _init():
        out_ref[...] = x2_ref[...]

    mb = m_ref[...]
    g = jnp.dot(mb, wg_ref[0], preferred_element_type=jnp.float32)
    u = jnp.dot(mb, wi_ref[0], preferred_element_type=jnp.float32)
    h = (g * jax.nn.sigmoid(g) * u).astype(jnp.bfloat16)
    y = jnp.dot(h, woe_ref[0], preferred_element_type=jnp.float32)
    comb = comb_ref[...]
    it = jax.lax.broadcasted_iota(jnp.int32, comb.shape, 1)
    w = jnp.sum(jnp.where(it == e, comb, 0.0), axis=1, keepdims=True)
    out_ref[...] += w * y


def kernel(x, attn_norm_w, Wq, Wk, Wv, Wo, q_norm_w, k_norm_w, ffn_norm_w,
           Wi, Wg, Woe, router_w, cos, sin):
    b, t, dim = x.shape
    nq = Wq.shape[1] // cos.shape[1]
    nkv = Wk.shape[1] // cos.shape[1]
    hd = cos.shape[1]
    ne, _, hid = Wi.shape
    blk = min(BLK, t)
    nt = t // blk

    x2d = x.reshape(t, dim)
    bf = jnp.bfloat16
    wq_b, wk_b, wv_b, wo_b = (w.astype(bf) for w in (Wq, Wk, Wv, Wo))
    wi_b, wg_b, woe_b = (w.astype(bf) for w in (Wi, Wg, Woe))

    p64 = _rot_perm(hd)
    pq = jnp.kron(jnp.eye(nq, dtype=jnp.float32), p64).astype(bf)
    pk = jnp.kron(jnp.eye(nkv, dtype=jnp.float32), p64).astype(bf)
    hq = jnp.kron(jnp.eye(nq, dtype=jnp.float32), jnp.ones((hd, 1), jnp.float32))
    hk = jnp.kron(jnp.eye(nkv, dtype=jnp.float32), jnp.ones((hd, 1), jnp.float32))
    cq = jnp.tile(cos, (1, nq))
    sq = jnp.tile(sin, (1, nq))
    ck = jnp.tile(cos, (1, nkv))
    sk = jnp.tile(sin, (1, nkv))
    qnw = jnp.tile(q_norm_w, (nq,)).reshape(1, nq * hd)
    knw = jnp.tile(k_norm_w, (nkv,)).reshape(1, nkv * hd)
    anw = attn_norm_w.reshape(1, dim)
    fnw = ffn_norm_w.reshape(1, dim)

    dq, dkv = nq * hd, nkv * hd

    full = lambda shape: pl.BlockSpec(shape, lambda i: (0,) * len(shape))
    rowblk = lambda w: pl.BlockSpec((blk, w), lambda i: (i, 0))

    q, k, v = pl.pallas_call(
        functools.partial(_prelude_body, hd=hd),
        grid=(nt,),
        in_specs=[
            rowblk(dim), full((1, dim)), full((dim, dq)), full((dim, dkv)),
            full((dim, dkv)), full((1, dq)), full((1, dkv)),
            rowblk(dq), rowblk(dq), rowblk(dkv), rowblk(dkv),
            full((dq, dq)), full((dkv, dkv)),
            full((dq, nq)), full((nq, dq)), full((dkv, nkv)), full((nkv, dkv)),
        ],
        out_specs=[rowblk(dq), rowblk(dkv), rowblk(dkv)],
        out_shape=[
            jax.ShapeDtypeStruct((t, dq), bf),
            jax.ShapeDtypeStruct((t, dkv), bf),
            jax.ShapeDtypeStruct((t, dkv), bf),
        ],
    )(x2d, anw, wq_b, wk_b, wv_b, qnw, knw, cq, sq, ck, sk,
      pq, pk, hq, hq.T, hk, hk.T)

    rep = nq // nkv
    q3 = q.reshape(t, nq, hd).transpose(1, 0, 2)
    k3 = k.reshape(t, nkv, hd).transpose(1, 0, 2)
    v3 = v.reshape(t, nkv, hd).transpose(1, 0, 2)
    ctx3 = pl.pallas_call(
        functools.partial(_attn_body, hd=hd),
        grid=(nq, nt),
        in_specs=[
            pl.BlockSpec((1, blk, hd), lambda h, i: (h, i, 0)),
            pl.BlockSpec((1, t, hd), lambda h, i: (h // rep, 0, 0)),
            pl.BlockSpec((1, t, hd), lambda h, i: (h // rep, 0, 0)),
        ],
        out_specs=pl.BlockSpec((1, blk, hd), lambda h, i: (h, i, 0)),
        out_shape=jax.ShapeDtypeStruct((nq, t, hd), bf),
    )(q3, k3, v3)
    ctx = ctx3.transpose(1, 0, 2).reshape(t, dq)

    x2, m, comb = pl.pallas_call(
        functools.partial(_post_body, ne=ne),
        grid=(nt,),
        in_specs=[rowblk(dq), full((dq, dim)), rowblk(dim), full((1, dim)),
                  full((dim, ne))],
        out_specs=[rowblk(dim), rowblk(dim), rowblk(ne)],
        out_shape=[
            jax.ShapeDtypeStruct((t, dim), jnp.float32),
            jax.ShapeDtypeStruct((t, dim), bf),
            jax.ShapeDtypeStruct((t, ne), jnp.float32),
        ],
    )(ctx, wo_b, x2d, fnw, router_w)

    out = pl.pallas_call(
        _ffn_body,
        grid=(nt, ne),
        in_specs=[
            pl.BlockSpec((blk, dim), lambda i, e: (i, 0)),
            pl.BlockSpec((1, dim, hid), lambda i, e: (e, 0, 0)),
            pl.BlockSpec((1, dim, hid), lambda i, e: (e, 0, 0)),
            pl.BlockSpec((1, hid, dim), lambda i, e: (e, 0, 0)),
            pl.BlockSpec((blk, ne), lambda i, e: (i, 0)),
            pl.BlockSpec((blk, dim), lambda i, e: (i, 0)),
        ],
        out_specs=pl.BlockSpec((blk, dim), lambda i, e: (i, 0)),
        out_shape=jax.ShapeDtypeStruct((t, dim), jnp.float32),
        compiler_params=pltpu.CompilerParams(
            dimension_semantics=("parallel", "arbitrary")),
    )(m, wg_b, wi_b, woe_b, comb, x2)

    return out.reshape(b, t, dim)
